# Initial kernel scaffold; baseline (speedup 1.0000x reference)
import jax
import jax.numpy as jnp
from jax.experimental import pallas as pl


def kernel(src_x, dst_x, W_src, b_src, W_dst, b_dst, W_as, b_as, W_ad, b_ad, W_a, b_a, W_o, b_o, skip, edge_index):
    raise NotImplementedError("write your pallas kernel here")


# speedup vs baseline: 15.1417x; 15.1417x over previous
import functools

import jax
import jax.numpy as jnp
from jax import lax
from jax.experimental import pallas as pl
from jax.experimental.pallas import tpu as pltpu
from jax.experimental.pallas import tpu_sc as plsc

N = 10000
E = 320000
HID = 128
OUT = 128
DPAD = 144

NC = 2
NS = 16
NW = NC * NS
EPW = E // NW
CHUNK = 80
NCHUNK = EPW // CHUNK
RPT = N // NS
LANES = 16


def _prep_body(src_x_ref, dst_x_ref, W_src_ref, b_src_ref, W_dst_ref,
               b_dst_ref, W_as_ref, b_as_ref, W_ad_ref, b_ad_ref, W_a_ref,
               b_a_ref, hid2_ref, hid_dst_ref, s_src_ref, s_dst_ref):
  src_x = src_x_ref[...]
  dst_x = dst_x_ref[...]

  def bn(h):
    mu = jnp.mean(h, axis=0, keepdims=True)
    d = h - mu
    var = jnp.mean(d * d, axis=0, keepdims=True)
    return d / jnp.sqrt(var + 1e-5)

  hs = bn(jnp.dot(src_x, W_src_ref[...], preferred_element_type=jnp.float32)
          + b_src_ref[...])
  hd = bn(jnp.dot(dst_x, W_dst_ref[...], preferred_element_type=jnp.float32)
          + b_dst_ref[...])
  hid_dst_ref[...] = hd
  pad = jnp.where(
      lax.broadcasted_iota(jnp.int32, (N, DPAD - HID), 1) == 0, 1.0, 0.0)
  hid2_ref[...] = jnp.concatenate([hs, pad], axis=1)

  w_lo = W_a_ref[0:HID, :]
  w_hi = W_a_ref[HID:2 * HID, :]
  v_s = jnp.dot(W_as_ref[...], w_hi, preferred_element_type=jnp.float32)
  v_d = jnp.dot(W_ad_ref[...], w_lo, preferred_element_type=jnp.float32)
  c = (jnp.dot(b_as_ref[...].reshape(1, HID), w_hi,
               preferred_element_type=jnp.float32)
       + jnp.dot(b_ad_ref[...].reshape(1, HID), w_lo,
                 preferred_element_type=jnp.float32)
       + b_a_ref[...].reshape(1, 1))
  s_src_ref[...] = jnp.dot(src_x, v_s, preferred_element_type=jnp.float32)
  s_dst_ref[...] = jnp.dot(dst_x, v_d,
                           preferred_element_type=jnp.float32) + c


_prep = pl.pallas_call(
    _prep_body,
    out_shape=[
        jax.ShapeDtypeStruct((N, DPAD), jnp.float32),
        jax.ShapeDtypeStruct((N, HID), jnp.float32),
        jax.ShapeDtypeStruct((N, 1), jnp.float32),
        jax.ShapeDtypeStruct((N, 1), jnp.float32),
    ],
)


def _sc_body(s_src_hbm, s_dst_hbm, sidx_hbm, didx_hbm, hid2_hbm, u_out_hbm,
             s_src_v, s_dst_v, sidx_v, didx_v, ex_v, rows_v, u_sh, sem):
  c = lax.axis_index("c")
  s = lax.axis_index("s")
  wid = s * NC + c

  def zero_row(e, carry2):
    for j in range(DPAD // LANES):
      rows_v[e, pl.ds(j * LANES, LANES)] = jnp.zeros((LANES,), jnp.float32)
    return carry2

  lax.fori_loop(0, CHUNK, zero_row, 0)
  for k in range(RPT // CHUNK):
    pltpu.sync_copy(rows_v, u_sh.at[pl.ds(s * RPT + k * CHUNK, CHUNK)])
  rem = RPT % CHUNK
  if rem:
    pltpu.sync_copy(rows_v.at[pl.ds(0, rem)],
                    u_sh.at[pl.ds(s * RPT + (RPT // CHUNK) * CHUNK, rem)])

  pltpu.sync_copy(s_src_hbm, s_src_v)
  pltpu.sync_copy(s_dst_hbm, s_dst_v)
  plsc.subcore_barrier()

  base0 = wid * EPW

  def chunk(i, carry):
    base = base0 + i * CHUNK
    pltpu.sync_copy(sidx_hbm.at[pl.ds(base, CHUNK)], sidx_v)
    pltpu.sync_copy(didx_hbm.at[pl.ds(base, CHUNK)], didx_v)
    gcopy = pltpu.async_copy(hid2_hbm.at[sidx_v], rows_v, sem)
    for i16 in range(CHUNK // LANES):
      si = sidx_v[pl.ds(i16 * LANES, LANES)]
      di = didx_v[pl.ds(i16 * LANES, LANES)]
      a = plsc.load_gather(s_src_v, [si]) + plsc.load_gather(s_dst_v, [di])
      e2a = jnp.exp(a + a)
      ta = 1.0 - 2.0 / (e2a + 1.0)
      ex_v[pl.ds(i16 * LANES, LANES)] = jnp.exp(ta)
    gcopy.wait()

    def scale_row(e, carry2):
      exs = plsc.load_gather(ex_v, [jnp.broadcast_to(e, (LANES,)).astype(jnp.int32)])
      for j in range(DPAD // LANES):
        rows_v[e, pl.ds(j * LANES, LANES)] = (
            rows_v[e, pl.ds(j * LANES, LANES)] * exs)
      return carry2

    lax.fori_loop(0, CHUNK, scale_row, 0)
    pltpu.sync_copy(rows_v, u_sh.at[didx_v], add=True)
    return carry

  lax.fori_loop(0, NCHUNK, chunk, 0)
  plsc.subcore_barrier()
  for k in range(RPT // CHUNK):
    pltpu.sync_copy(u_sh.at[pl.ds(s * RPT + k * CHUNK, CHUNK)],
                    u_out_hbm.at[c, pl.ds(s * RPT + k * CHUNK, CHUNK)])
  rem = RPT % CHUNK
  if rem:
    off = (RPT // CHUNK) * CHUNK
    pltpu.sync_copy(u_sh.at[pl.ds(s * RPT + off, rem)],
                    u_out_hbm.at[c, pl.ds(s * RPT + off, rem)])


_sc_edge = functools.partial(
    pl.kernel,
    out_type=jax.ShapeDtypeStruct((NC, N, DPAD), jnp.float32),
    mesh=plsc.VectorSubcoreMesh(core_axis_name="c", subcore_axis_name="s"),
    scratch_types=[
        pltpu.VMEM((N,), jnp.float32),
        pltpu.VMEM((N,), jnp.float32),
        pltpu.VMEM((CHUNK,), jnp.int32),
        pltpu.VMEM((CHUNK,), jnp.int32),
        pltpu.VMEM((CHUNK,), jnp.float32),
        pltpu.VMEM((CHUNK, DPAD), jnp.float32),
        pltpu.VMEM_SHARED((N, DPAD), jnp.float32),
        pltpu.SemaphoreType.DMA,
    ],
    compiler_params=pltpu.CompilerParams(use_tc_tiling_on_sc=False,
                                         needs_layout_passes=False),
)(_sc_body)


def _post_body(u_ref, hid_dst_ref, W_o_ref, b_o_ref, skip_ref, out_ref):
  u = u_ref[0] + u_ref[1]
  t = u[:, 0:HID] / u[:, HID:HID + 1]
  alpha = 1.0 / (1.0 + jnp.exp(-skip_ref[...].reshape(1, 1)))
  trans = alpha * t + (1.0 - alpha) * hid_dst_ref[...]
  out_ref[...] = jnp.dot(trans, W_o_ref[...],
                         preferred_element_type=jnp.float32) + b_o_ref[...]


_post = pl.pallas_call(
    _post_body,
    out_shape=jax.ShapeDtypeStruct((N, OUT), jnp.float32),
)


def kernel(src_x, dst_x, W_src, b_src, W_dst, b_dst, W_as, b_as, W_ad, b_ad,
           W_a, b_a, W_o, b_o, skip, edge_index):
  hid2, hid_dst, s_src, s_dst = _prep(src_x, dst_x, W_src, b_src, W_dst,
                                      b_dst, W_as, b_as, W_ad, b_ad, W_a, b_a)
  u = _sc_edge(s_src.reshape(N), s_dst.reshape(N), edge_index[0],
               edge_index[1], hid2)
  return _post(u, hid_dst, W_o, b_o, skip)


# speedup vs baseline: 31.2277x; 2.0624x over previous
import functools

import jax
import jax.numpy as jnp
from jax import lax
from jax.experimental import pallas as pl
from jax.experimental.pallas import tpu as pltpu
from jax.experimental.pallas import tpu_sc as plsc

N = 10000
E = 320000
HID = 128
OUT = 128

NC = 2
NS = 16
NW = NC * NS
EPW = E // NW
CHUNK = 125
NCHUNK = EPW // CHUNK
RPT = N // NS
LANES = 16


def _scal_body(src_x_ref, dst_x_ref, W_as_ref, b_as_ref, W_ad_ref, b_ad_ref,
               W_a_ref, b_a_ref, s_src_ref, s_dst_ref):
  w_lo = W_a_ref[0:HID, :]
  w_hi = W_a_ref[HID:2 * HID, :]
  v_s = jnp.dot(W_as_ref[...], w_hi, preferred_element_type=jnp.float32)
  v_d = jnp.dot(W_ad_ref[...], w_lo, preferred_element_type=jnp.float32)
  c = (jnp.dot(b_as_ref[...].reshape(1, HID), w_hi,
               preferred_element_type=jnp.float32)
       + jnp.dot(b_ad_ref[...].reshape(1, HID), w_lo,
                 preferred_element_type=jnp.float32)
       + b_a_ref[...].reshape(1, 1))
  s_src_ref[...] = jnp.dot(src_x_ref[...], v_s,
                           preferred_element_type=jnp.float32)
  s_dst_ref[...] = jnp.dot(dst_x_ref[...], v_d,
                           preferred_element_type=jnp.float32) + c


_scal = pl.pallas_call(
    _scal_body,
    out_shape=[
        jax.ShapeDtypeStruct((N, 1), jnp.float32),
        jax.ShapeDtypeStruct((N, 1), jnp.float32),
    ],
)


def _hid_body(src_x_ref, dst_x_ref, W_src_ref, b_src_ref, W_dst_ref,
              b_dst_ref, hid_src_ref, hid_dst_ref):
  def bn(h):
    mu = jnp.mean(h, axis=0, keepdims=True)
    d = h - mu
    var = jnp.mean(d * d, axis=0, keepdims=True)
    return d / jnp.sqrt(var + 1e-5)

  hid_src_ref[...] = bn(
      jnp.dot(src_x_ref[...], W_src_ref[...],
              preferred_element_type=jnp.float32) + b_src_ref[...])
  hid_dst_ref[...] = bn(
      jnp.dot(dst_x_ref[...], W_dst_ref[...],
              preferred_element_type=jnp.float32) + b_dst_ref[...])


_hid = pl.pallas_call(
    _hid_body,
    out_shape=[
        jax.ShapeDtypeStruct((N, HID), jnp.float32),
        jax.ShapeDtypeStruct((N, HID), jnp.float32),
    ],
)


def _exw_body(s_src_hbm, s_dst_hbm, eidx_hbm, ex_out_hbm, den_out_hbm,
              s_src_v, s_dst_v, sidx_v, didx_v, ex_v, den_v):
  c = lax.axis_index("c")
  s = lax.axis_index("s")
  wid = s * NC + c
  pltpu.sync_copy(s_src_hbm, s_src_v)
  pltpu.sync_copy(s_dst_hbm, s_dst_v)
  pltpu.sync_copy(eidx_hbm.at[0, wid], sidx_v)
  pltpu.sync_copy(eidx_hbm.at[1, wid], didx_v)

  def zstep(k, carry):
    den_v[pl.ds(k * LANES, LANES)] = jnp.zeros((LANES,), jnp.float32)
    return carry

  lax.fori_loop(0, N // LANES, zstep, 0)

  def step(k, carry):
    si = sidx_v[pl.ds(k * LANES, LANES)]
    di = didx_v[pl.ds(k * LANES, LANES)]
    a = plsc.load_gather(s_src_v, [si]) + plsc.load_gather(s_dst_v, [di])
    e2a = jnp.exp(a + a)
    ta = 1.0 - 2.0 / (e2a + 1.0)
    ex = jnp.exp(ta)
    ex_v[pl.ds(k * LANES, LANES)] = ex
    plsc.addupdate_scatter(den_v, [di], ex)
    return carry

  lax.fori_loop(0, EPW // LANES, step, 0)
  pltpu.sync_copy(ex_v, ex_out_hbm.at[wid])
  pltpu.sync_copy(den_v, den_out_hbm.at[wid])


_exw = functools.partial(
    pl.kernel,
    out_type=[
        jax.ShapeDtypeStruct((NW, EPW), jnp.float32),
        jax.ShapeDtypeStruct((NW, N), jnp.float32),
    ],
    mesh=plsc.VectorSubcoreMesh(core_axis_name="c", subcore_axis_name="s"),
    scratch_types=[
        pltpu.VMEM((N,), jnp.float32),
        pltpu.VMEM((N,), jnp.float32),
        pltpu.VMEM((EPW,), jnp.int32),
        pltpu.VMEM((EPW,), jnp.int32),
        pltpu.VMEM((EPW,), jnp.float32),
        pltpu.VMEM((N,), jnp.float32),
    ],
    compiler_params=pltpu.CompilerParams(use_tc_tiling_on_sc=False,
                                         needs_layout_passes=False),
)(_exw_body)


def _agg_body(eidx_hbm, ex_hbm, hid_hbm, u_out_hbm, sd_v, ex4_v, rows_v,
              u_sh, isem0, isem1, isem2, isem3, gsem0, gsem1, ssem0, ssem1):
  c = lax.axis_index("c")
  s = lax.axis_index("s")
  wid = s * NC + c
  isems = (isem0, isem1, isem2, isem3)
  gsems = (gsem0, gsem1)
  ssems = (ssem0, ssem1)

  def zero_row(e, carry2):
    for j in range(HID // LANES):
      rows_v[0, e, pl.ds(j * LANES, LANES)] = jnp.zeros((LANES,), jnp.float32)
    return carry2

  lax.fori_loop(0, CHUNK, zero_row, 0)
  for k in range(RPT // CHUNK):
    pltpu.sync_copy(rows_v.at[0], u_sh.at[pl.ds(s * RPT + k * CHUNK, CHUNK)])
  rem = RPT % CHUNK
  if rem:
    pltpu.sync_copy(rows_v.at[0, pl.ds(0, rem)],
                    u_sh.at[pl.ds(s * RPT + (RPT // CHUNK) * CHUNK, rem)])
  plsc.subcore_barrier()

  def idx_start(t, slot):
    pltpu.async_copy(eidx_hbm.at[0, wid, t], sd_v.at[slot, 0], isems[slot])
    pltpu.async_copy(eidx_hbm.at[1, wid, t], sd_v.at[slot, 1], isems[slot])
    pltpu.async_copy(ex_hbm.at[wid, t], ex4_v.at[slot], isems[slot])

  def idx_wait(t, slot):
    pltpu.make_async_copy(eidx_hbm.at[0, wid, t], sd_v.at[slot, 0],
                          isems[slot]).wait()
    pltpu.make_async_copy(eidx_hbm.at[1, wid, t], sd_v.at[slot, 1],
                          isems[slot]).wait()
    pltpu.make_async_copy(ex_hbm.at[wid, t], ex4_v.at[slot],
                          isems[slot]).wait()

  def gather_start(t, slot, rslot):
    pltpu.async_copy(hid_hbm.at[sd_v.at[slot, 0]], rows_v.at[rslot],
                     gsems[rslot])

  def gather_wait(t, slot, rslot):
    pltpu.make_async_copy(hid_hbm.at[sd_v.at[slot, 0]], rows_v.at[rslot],
                          gsems[rslot]).wait()

  def scatter_start(t, slot, rslot):
    pltpu.async_copy(rows_v.at[rslot], u_sh.at[sd_v.at[slot, 1]],
                     ssems[rslot], add=True)

  def scatter_wait(t, slot, rslot):
    pltpu.make_async_copy(rows_v.at[rslot], u_sh.at[sd_v.at[slot, 1]],
                          ssems[rslot]).wait()

  idx_start(0, 0)
  idx_start(1, 1)
  idx_wait(0, 0)
  gather_start(0, 0, 0)

  def body(i, carry):
    for p in range(4):
      t = i * 4 + p
      r = p % 2
      gather_wait(t, p, r)

      @pl.when(t + 1 < NCHUNK)
      def _():
        idx_wait(t + 1, (p + 1) % 4)

      @pl.when(t >= 1)
      def _():
        scatter_wait(t - 1, (p + 3) % 4, 1 - r)

      @pl.when(t + 1 < NCHUNK)
      def _():
        gather_start(t + 1, (p + 1) % 4, 1 - r)

      @pl.when(t + 2 < NCHUNK)
      def _():
        idx_start(t + 2, (p + 2) % 4)

      def _scale(e, carry3):
        exs = plsc.load_gather(
            ex4_v, [jnp.full((LANES,), p, jnp.int32),
                    jnp.broadcast_to(e, (LANES,)).astype(jnp.int32)])
        for j in range(HID // LANES):
          rows_v[r, e, pl.ds(j * LANES, LANES)] = (
              rows_v[r, e, pl.ds(j * LANES, LANES)] * exs)
        return carry3

      lax.fori_loop(0, CHUNK, _scale, 0)

      scatter_start(t, p, r)
    return carry

  lax.fori_loop(0, NCHUNK // 4, body, 0)
  scatter_wait(NCHUNK - 1, (NCHUNK - 1) % 4, (NCHUNK - 1) % 2)
  plsc.subcore_barrier()
  for k in range(RPT // CHUNK):
    pltpu.sync_copy(u_sh.at[pl.ds(s * RPT + k * CHUNK, CHUNK)],
                    u_out_hbm.at[c, pl.ds(s * RPT + k * CHUNK, CHUNK)])
  if rem:
    off = (RPT // CHUNK) * CHUNK
    pltpu.sync_copy(u_sh.at[pl.ds(s * RPT + off, rem)],
                    u_out_hbm.at[c, pl.ds(s * RPT + off, rem)])


_agg = functools.partial(
    pl.kernel,
    out_type=jax.ShapeDtypeStruct((NC, N, HID), jnp.float32),
    mesh=plsc.VectorSubcoreMesh(core_axis_name="c", subcore_axis_name="s"),
    scratch_types=[
        pltpu.VMEM((4, 2, CHUNK), jnp.int32),
        pltpu.VMEM((4, CHUNK), jnp.float32),
        pltpu.VMEM((2, CHUNK, HID), jnp.float32),
        pltpu.VMEM_SHARED((N, HID), jnp.float32),
        pltpu.SemaphoreType.DMA,
        pltpu.SemaphoreType.DMA,
        pltpu.SemaphoreType.DMA,
        pltpu.SemaphoreType.DMA,
        pltpu.SemaphoreType.DMA,
        pltpu.SemaphoreType.DMA,
        pltpu.SemaphoreType.DMA,
        pltpu.SemaphoreType.DMA,
    ],
    compiler_params=pltpu.CompilerParams(use_tc_tiling_on_sc=False,
                                         needs_layout_passes=False),
)(_agg_body)


def _post_body(u_ref, den_ref, hid_dst_ref, W_o_ref, b_o_ref, skip_ref,
               out_ref):
  u = u_ref[0] + u_ref[1]
  den = jnp.sum(den_ref[...], axis=0)
  t = u / den[:, None]
  alpha = 1.0 / (1.0 + jnp.exp(-skip_ref[...].reshape(1, 1)))
  trans = alpha * t + (1.0 - alpha) * hid_dst_ref[...]
  out_ref[...] = jnp.dot(trans, W_o_ref[...],
                         preferred_element_type=jnp.float32) + b_o_ref[...]


_post = pl.pallas_call(
    _post_body,
    out_shape=jax.ShapeDtypeStruct((N, OUT), jnp.float32),
)


def kernel(src_x, dst_x, W_src, b_src, W_dst, b_dst, W_as, b_as, W_ad, b_ad,
           W_a, b_a, W_o, b_o, skip, edge_index):
  s_src, s_dst = _scal(src_x, dst_x, W_as, b_as, W_ad, b_ad, W_a, b_a)
  hid_src, hid_dst = _hid(src_x, dst_x, W_src, b_src, W_dst, b_dst)
  eidx = edge_index.reshape(2, NW, EPW)
  ex, den = _exw(s_src.reshape(N), s_dst.reshape(N), eidx)
  u = _agg(edge_index.reshape(2, NW, NCHUNK, CHUNK),
           ex.reshape(NW, NCHUNK, CHUNK), hid_src)
  return _post(u, den, hid_dst, W_o, b_o, skip)
